# Initial kernel scaffold; baseline (speedup 1.0000x reference)
import jax
import jax.numpy as jnp
from jax.experimental import pallas as pl


def kernel(x, pos, edge_index, batch, emb_W, emb_b, c1_W1, c1_b1, c1_g, c1_be, c1_W2, c1_b2, c2_W1, c2_b1, c2_g, c2_be, c2_W2, c2_b2, c3_W1, c3_b1, c3_g, c3_be, c3_W2, c3_b2, l1_W, l1_b, l2_W, l2_b):
    raise NotImplementedError("write your pallas kernel here")


# speedup vs baseline: 3.7389x; 3.7389x over previous
import functools

import jax
import jax.numpy as jnp
from jax import lax
from jax.experimental import pallas as pl
from jax.experimental.pallas import tpu as pltpu
from jax.experimental.pallas import tpu_sc as plsc

N = 50000
E = 800000
F = 128
H = 64
G = 64
C = 16

BLK = 2000
NBLK = N // BLK

NC = 2
NS = 16
HALF = N // NC
CHUNK = 128
NCH = E // CHUNK
TRASH_PER_TILE = CHUNK
R_PAD = 27136
NZ = R_PAD // CHUNK
WB = 40
NWB = HALF // WB


def _sc_agg_body(h_hbm, src_hbm, dst_hbm, out_hbm,
                 src_v, dstr_v, scat_v, rows_v, wb_v, agg_sh, sem):
    c = lax.axis_index("c")
    s = lax.axis_index("s")
    base_node = c * HALF
    iota16 = lax.broadcasted_iota(jnp.int32, (16,), 0)

    def _zrow(i, carry):
        for k in range(H // 16):
            rows_v[i, pl.ds(k * 16, 16)] = jnp.zeros((16,), jnp.float32)
        return carry
    lax.fori_loop(0, CHUNK, _zrow, 0)

    n_z = (NZ - s + NS - 1) // NS
    def _zchunk(i, carry):
        j = s + i * NS
        pltpu.sync_copy(rows_v, agg_sh.at[pl.ds(j * CHUNK, CHUNK)])
        return carry
    lax.fori_loop(0, n_z, _zchunk, 0)

    plsc.subcore_barrier()

    n_m = (NCH - s + NS - 1) // NS
    def _chunk(i, carry):
        base_e = (s + i * NS) * CHUNK
        pltpu.sync_copy(src_hbm.at[pl.ds(base_e, CHUNK)], src_v)
        pltpu.sync_copy(dst_hbm.at[pl.ds(base_e, CHUNK)], dstr_v)
        for j in range(CHUNK // 16):
            d = dstr_v[pl.ds(j * 16, 16)]
            loc = d - base_node
            ok = (loc >= 0) & (loc < HALF)
            trash = HALF + s * TRASH_PER_TILE + j * 16 + iota16
            scat_v[pl.ds(j * 16, 16)] = jnp.where(ok, loc, trash)
        pltpu.async_copy(h_hbm.at[src_v], rows_v, sem).wait()
        pltpu.sync_copy(rows_v, agg_sh.at[scat_v], add=True)
        return carry
    lax.fori_loop(0, n_m, _chunk, 0)

    plsc.subcore_barrier()

    n_w = (NWB - s + NS - 1) // NS
    def _wb(i, carry):
        j = s + i * NS
        pltpu.sync_copy(agg_sh.at[pl.ds(j * WB, WB)], wb_v)
        pltpu.sync_copy(wb_v, out_hbm.at[pl.ds(base_node + j * WB, WB)])
        return carry
    lax.fori_loop(0, n_w, _wb, 0)


@functools.cache
def _get_sc_agg():
    return pl.kernel(
        _sc_agg_body,
        out_type=jax.ShapeDtypeStruct((N, H), jnp.float32),
        mesh=plsc.VectorSubcoreMesh(core_axis_name="c", subcore_axis_name="s",
                                    num_cores=NC, num_subcores=NS),
        compiler_params=pltpu.CompilerParams(use_tc_tiling_on_sc=False),
        scratch_types=[
            pltpu.VMEM((CHUNK,), jnp.int32),
            pltpu.VMEM((CHUNK,), jnp.int32),
            pltpu.VMEM((CHUNK,), jnp.int32),
            pltpu.VMEM((CHUNK, H), jnp.float32),
            pltpu.VMEM((WB, H), jnp.float32),
            pltpu.VMEM_SHARED((R_PAD, H), jnp.float32),
            pltpu.SemaphoreType.DMA,
        ],
    )


def _sc_agg(h, src, dst):
    return _get_sc_agg()(h, src, dst)


def _embed_body(x_ref, w_ref, b_ref, o_ref):
    o_ref[...] = (
        jnp.dot(x_ref[...], w_ref[...], preferred_element_type=jnp.float32)
        + b_ref[...]
    )


def _embed(x, w, b):
    return pl.pallas_call(
        _embed_body,
        grid=(NBLK,),
        in_specs=[
            pl.BlockSpec((BLK, F), lambda i: (i, 0)),
            pl.BlockSpec((F, H), lambda i: (0, 0)),
            pl.BlockSpec((1, H), lambda i: (0, 0)),
        ],
        out_specs=pl.BlockSpec((BLK, H), lambda i: (i, 0)),
        out_shape=jax.ShapeDtypeStruct((N, H), jnp.float32),
    )(x, w, b.reshape(1, H))


def _pre_body(h_ref, a_ref, w_ref, b_ref, t_ref, s_ref):
    i = pl.program_id(0)
    u = h_ref[...] + a_ref[...]
    t = jnp.dot(u, w_ref[...], preferred_element_type=jnp.float32) + b_ref[...]
    t_ref[...] = t

    @pl.when(i == 0)
    def _():
        s_ref[...] = jnp.zeros_like(s_ref)

    s_ref[0:1, :] += jnp.sum(t, axis=0, keepdims=True)
    s_ref[1:2, :] += jnp.sum(t * t, axis=0, keepdims=True)


def _pre(h, agg, w1, b1):
    return pl.pallas_call(
        _pre_body,
        grid=(NBLK,),
        in_specs=[
            pl.BlockSpec((BLK, H), lambda i: (i, 0)),
            pl.BlockSpec((BLK, H), lambda i: (i, 0)),
            pl.BlockSpec((H, H), lambda i: (0, 0)),
            pl.BlockSpec((1, H), lambda i: (0, 0)),
        ],
        out_specs=[
            pl.BlockSpec((BLK, H), lambda i: (i, 0)),
            pl.BlockSpec((8, H), lambda i: (0, 0)),
        ],
        out_shape=[
            jax.ShapeDtypeStruct((N, H), jnp.float32),
            jax.ShapeDtypeStruct((8, H), jnp.float32),
        ],
    )(h, agg, w1, b1.reshape(1, H))


def _post_body(t_ref, s_ref, g_ref, be_ref, w_ref, b_ref, o_ref):
    m = s_ref[0:1, :] * (1.0 / N)
    v = s_ref[1:2, :] * (1.0 / N) - m * m
    scale = lax.rsqrt(v + 1e-5) * g_ref[...]
    z = jnp.maximum((t_ref[...] - m) * scale + be_ref[...], 0.0)
    o_ref[...] = jnp.maximum(
        jnp.dot(z, w_ref[...], preferred_element_type=jnp.float32) + b_ref[...],
        0.0,
    )


def _post(t, stats, g, be, w2, b2):
    return pl.pallas_call(
        _post_body,
        grid=(NBLK,),
        in_specs=[
            pl.BlockSpec((BLK, H), lambda i: (i, 0)),
            pl.BlockSpec((8, H), lambda i: (0, 0)),
            pl.BlockSpec((1, H), lambda i: (0, 0)),
            pl.BlockSpec((1, H), lambda i: (0, 0)),
            pl.BlockSpec((H, H), lambda i: (0, 0)),
            pl.BlockSpec((1, H), lambda i: (0, 0)),
        ],
        out_specs=pl.BlockSpec((BLK, H), lambda i: (i, 0)),
        out_shape=jax.ShapeDtypeStruct((N, H), jnp.float32),
    )(t, stats, g.reshape(1, H), be.reshape(1, H), w2, b2.reshape(1, H))


def _pool_body(b_ref, h1_ref, h2_ref, h3_ref, w1_ref, b1_ref, w2_ref, b2_ref,
               ge_ref, out_ref, cnt_ref):
    i = pl.program_id(0)

    @pl.when(i == 0)
    def _():
        ge_ref[...] = jnp.zeros_like(ge_ref)
        out_ref[...] = jnp.zeros_like(out_ref)
        cnt_ref[...] = jnp.zeros_like(cnt_ref)

    bt = b_ref[0, 0, :]
    onehot = (
        bt[:, None] == lax.broadcasted_iota(jnp.int32, (1, G), 1)
    ).astype(jnp.float32)
    hcat = jnp.concatenate([h1_ref[...], h2_ref[...], h3_ref[...]], axis=1)
    ge_ref[...] += lax.dot_general(
        onehot, hcat, (((0,), (0,)), ((), ())),
        preferred_element_type=jnp.float32)
    cnt_ref[...] += lax.dot_general(
        onehot, jnp.ones((BLK, 1), jnp.float32), (((0,), (0,)), ((), ())),
        preferred_element_type=jnp.float32)

    @pl.when(i == NBLK - 1)
    def _():
        ge = ge_ref[...] / jnp.maximum(cnt_ref[...], 1.0)
        ge_ref[...] = ge
        hh = jnp.maximum(
            jnp.dot(ge, w1_ref[...], preferred_element_type=jnp.float32)
            + b1_ref[...],
            0.0,
        )
        out_ref[...] = (
            jnp.dot(hh, w2_ref[...], preferred_element_type=jnp.float32)
            + b2_ref[...]
        )


def _pool_head(batch3d, h1, h2, h3, l1_W, l1_b, l2_W, l2_b):
    return pl.pallas_call(
        _pool_body,
        grid=(NBLK,),
        in_specs=[
            pl.BlockSpec((1, 1, BLK), lambda i: (i, 0, 0)),
            pl.BlockSpec((BLK, H), lambda i: (i, 0)),
            pl.BlockSpec((BLK, H), lambda i: (i, 0)),
            pl.BlockSpec((BLK, H), lambda i: (i, 0)),
            pl.BlockSpec((3 * H, 3 * H), lambda i: (0, 0)),
            pl.BlockSpec((1, 3 * H), lambda i: (0, 0)),
            pl.BlockSpec((3 * H, C), lambda i: (0, 0)),
            pl.BlockSpec((1, C), lambda i: (0, 0)),
        ],
        out_specs=[
            pl.BlockSpec((G, 3 * H), lambda i: (0, 0)),
            pl.BlockSpec((G, C), lambda i: (0, 0)),
        ],
        out_shape=[
            jax.ShapeDtypeStruct((G, 3 * H), jnp.float32),
            jax.ShapeDtypeStruct((G, C), jnp.float32),
        ],
        scratch_shapes=[pltpu.VMEM((G, 1), jnp.float32)],
    )(batch3d, h1, h2, h3, l1_W, l1_b.reshape(1, 3 * H), l2_W,
      l2_b.reshape(1, C))


def kernel(x, pos, edge_index, batch, emb_W, emb_b,
           c1_W1, c1_b1, c1_g, c1_be, c1_W2, c1_b2,
           c2_W1, c2_b1, c2_g, c2_be, c2_W2, c2_b2,
           c3_W1, c3_b1, c3_g, c3_be, c3_W2, c3_b2,
           l1_W, l1_b, l2_W, l2_b):
    src = edge_index[0]
    dst = edge_index[1]
    batch3d = batch.reshape(NBLK, 1, BLK)

    h = _embed(x, emb_W, emb_b)

    hs = []
    for (W1, b1, g, be, W2, b2) in (
        (c1_W1, c1_b1, c1_g, c1_be, c1_W2, c1_b2),
        (c2_W1, c2_b1, c2_g, c2_be, c2_W2, c2_b2),
        (c3_W1, c3_b1, c3_g, c3_be, c3_W2, c3_b2),
    ):
        agg = _sc_agg(h, src, dst)
        t, stats = _pre(h, agg, W1, b1)
        h = _post(t, stats, g, be, W2, b2)
        hs.append(h)

    graph_emb, out = _pool_head(batch3d, hs[0], hs[1], hs[2],
                                l1_W, l1_b, l2_W, l2_b)
    return (graph_emb, out)


# speedup vs baseline: 12.3973x; 3.3158x over previous
import functools

import jax
import jax.numpy as jnp
from jax import lax
from jax.experimental import pallas as pl
from jax.experimental.pallas import tpu as pltpu
from jax.experimental.pallas import tpu_sc as plsc

N = 50000
E = 800000
F = 128
H = 64
G = 64
C = 16

BLK = 2000
NBLK = N // BLK

NC = 2
NS = 16
HALF = N // NC
CHUNK = 128
NCH = E // CHUNK
TRASH_PER_TILE = 16
R_PAD = 25344
NZ = R_PAD // CHUNK
WB = 40
NWB = HALF // WB


EPT = E // NS
PBLK = 2000
CAPR = 400
CAPE = CAPR * CHUNK
STG = 2048
RING = 2


def _sc_part_body(src_hbm, dst_hbm, srcc_hbm, dstc_hbm, kcnt_hbm,
                  stg_s, stg_d, pend_s, pend_d, kv, sem):
    c = lax.axis_index("c")
    s = lax.axis_index("s")
    base_node = c * HALF
    iota16 = lax.broadcasted_iota(jnp.int32, (16,), 0)
    ebase = s * EPT

    def _blk(bi, cnt):
        pltpu.sync_copy(src_hbm.at[pl.ds(ebase + bi * PBLK, PBLK)], stg_s)
        pltpu.sync_copy(dst_hbm.at[pl.ds(ebase + bi * PBLK, PBLK)], stg_d)

        def _grp(g, cnt):
            sv = stg_s[pl.ds(g * 16, 16)]
            dv = stg_d[pl.ds(g * 16, 16)]
            loc = dv - base_node
            ok = plsc.bitcast(loc, jnp.uint32) < jnp.uint32(HALF)
            plsc.store_compressed(pend_s.at[pl.ds(cnt, 16)], sv, mask=ok)
            plsc.store_compressed(pend_d.at[pl.ds(cnt, 16)], loc, mask=ok)
            return cnt + jnp.sum(ok.astype(jnp.int32))

        return lax.fori_loop(0, PBLK // 16, _grp, cnt)

    cnt = lax.fori_loop(0, EPT // PBLK, _blk, jnp.int32(0))

    for t in range(8):
        pend_s[pl.ds(cnt + t * 16, 16)] = s * 128 + t * 16 + iota16
        pend_d[pl.ds(cnt + t * 16, 16)] = (
            HALF + s * TRASH_PER_TILE + t * 16 + iota16)

    k = (cnt + CHUNK - 1) // CHUNK
    kv[...] = jnp.full((16,), 0, jnp.int32) + k
    pltpu.sync_copy(kv, kcnt_hbm.at[c, s])

    rb = (c * NS + s) * CAPE
    n_w = (k + 15) // 16
    def _wb(i, carry):
        pltpu.sync_copy(pend_s.at[pl.ds(i * STG, STG)],
                        srcc_hbm.at[pl.ds(rb + i * STG, STG)])
        pltpu.sync_copy(pend_d.at[pl.ds(i * STG, STG)],
                        dstc_hbm.at[pl.ds(rb + i * STG, STG)])
        return carry
    lax.fori_loop(0, n_w, _wb, 0)


@functools.cache
def _get_sc_partition():
    return pl.kernel(
        _sc_part_body,
        out_type=(
            jax.ShapeDtypeStruct((NC * NS * CAPE,), jnp.int32),
            jax.ShapeDtypeStruct((NC * NS * CAPE,), jnp.int32),
            jax.ShapeDtypeStruct((NC, NS, 16), jnp.int32),
        ),
        mesh=plsc.VectorSubcoreMesh(core_axis_name="c", subcore_axis_name="s",
                                    num_cores=NC, num_subcores=NS),
        compiler_params=pltpu.CompilerParams(use_tc_tiling_on_sc=False,
                                             needs_layout_passes=False),
        scratch_types=[
            pltpu.VMEM((PBLK,), jnp.int32),
            pltpu.VMEM((PBLK,), jnp.int32),
            pltpu.VMEM((CAPE,), jnp.int32),
            pltpu.VMEM((CAPE,), jnp.int32),
            pltpu.VMEM((16,), jnp.int32),
            pltpu.SemaphoreType.DMA,
        ],
    )


def _sc_agg2_body(h_hbm, srcc_hbm, dstc_hbm, kcnt_hbm, out_hbm,
                  stg_s0, stg_d0, stg_s1, stg_d1, rows, scat, kv, wb_v,
                  agg_sh, sem_g, sem_st):
    c = lax.axis_index("c")
    s = lax.axis_index("s")
    base_node = c * HALF

    def _zrow(i, carry):
        for t in range(H // 16):
            rows[0, i, pl.ds(t * 16, 16)] = jnp.zeros((16,), jnp.float32)
        return carry
    lax.fori_loop(0, CHUNK, _zrow, 0)

    n_z = (NZ - s + NS - 1) // NS
    def _zchunk(i, carry):
        pltpu.sync_copy(rows.at[0], agg_sh.at[pl.ds((s + i * NS) * CHUNK,
                                                    CHUNK)])
        return carry
    lax.fori_loop(0, n_z, _zchunk, 0)

    plsc.subcore_barrier()

    pltpu.sync_copy(kcnt_hbm.at[c, s], kv)
    k = jnp.max(kv[...])
    nsb = (k + 15) // 16
    rb = (c * NS + s) * CAPE

    def _stage(buf_s, buf_d, sb):
        pltpu.async_copy(srcc_hbm.at[pl.ds(rb + sb * STG, STG)], buf_s,
                         sem_st)
        pltpu.async_copy(dstc_hbm.at[pl.ds(rb + sb * STG, STG)], buf_d,
                         sem_st)

    def _stage_wait(buf_s, buf_d):
        pltpu.make_async_copy(srcc_hbm.at[pl.ds(0, STG)], buf_s,
                              sem_st).wait()
        pltpu.make_async_copy(dstc_hbm.at[pl.ds(0, STG)], buf_d,
                              sem_st).wait()

    def _fire(buf_s, off, r, m):
        @pl.when(m < k)
        def _():
            pltpu.async_copy(h_hbm.at[buf_s.at[pl.ds(off * CHUNK, CHUNK)]],
                             rows.at[r], sem_g)

    pltpu.sync_copy(srcc_hbm.at[pl.ds(rb, STG)], stg_s0)
    pltpu.sync_copy(dstc_hbm.at[pl.ds(rb, STG)], stg_d0)
    for m in range(RING):
        _fire(stg_s0, m, m % RING, m)

    npair = (nsb + 1) // 2
    def _pair(p, carry):
        jbase = p * 32

        @pl.when(2 * p + 1 < nsb)
        def _():
            _stage(stg_s1, stg_d1, 2 * p + 1)

        for b in range(32):
            if b == 16:
                @pl.when(2 * p + 2 < nsb)
                def _():
                    _stage(stg_s0, stg_d0, 2 * p + 2)
            if b == 11:
                @pl.when(2 * p + 1 < nsb)
                def _():
                    _stage_wait(stg_s1, stg_d1)
            if b == 27:
                @pl.when(2 * p + 2 < nsb)
                def _():
                    _stage_wait(stg_s0, stg_d0)

            dbuf, off = (stg_d0, b) if b < 16 else (stg_d1, b - 16)
            r = b % RING
            j = jbase + b

            @pl.when(j < k)
            def _():
                pltpu.make_async_copy(h_hbm.at[pl.ds(0, CHUNK)], rows.at[r],
                                      sem_g).wait()
                for g in range(CHUNK // 16):
                    scat[r, pl.ds(g * 16, 16)] = (
                        dbuf[pl.ds(off * CHUNK + g * 16, 16)])
                pltpu.sync_copy(rows.at[r], agg_sh.at[scat.at[r]], add=True)

            bb = b + RING
            if bb < 32:
                nbuf, noff = (stg_s0, bb) if bb < 16 else (stg_s1, bb - 16)
            else:
                nbuf, noff = stg_s0, bb - 32
            _fire(nbuf, noff, bb % RING, j + RING)
        return carry
    lax.fori_loop(0, npair, _pair, 0)

    plsc.subcore_barrier()

    n_w = (NWB - s + NS - 1) // NS
    def _wb(i, carry):
        jj = s + i * NS
        pltpu.sync_copy(agg_sh.at[pl.ds(jj * WB, WB)], wb_v)
        pltpu.sync_copy(wb_v, out_hbm.at[pl.ds(base_node + jj * WB, WB)])
        return carry
    lax.fori_loop(0, n_w, _wb, 0)


@functools.cache
def _get_sc_agg():
    return pl.kernel(
        _sc_agg2_body,
        out_type=jax.ShapeDtypeStruct((N, H), jnp.float32),
        mesh=plsc.VectorSubcoreMesh(core_axis_name="c", subcore_axis_name="s",
                                    num_cores=NC, num_subcores=NS),
        compiler_params=pltpu.CompilerParams(use_tc_tiling_on_sc=False,
                                             needs_layout_passes=False),
        scratch_types=[
            pltpu.VMEM((STG,), jnp.int32),
            pltpu.VMEM((STG,), jnp.int32),
            pltpu.VMEM((STG,), jnp.int32),
            pltpu.VMEM((STG,), jnp.int32),
            pltpu.VMEM((RING, CHUNK, H), jnp.float32),
            pltpu.VMEM((RING, CHUNK), jnp.int32),
            pltpu.VMEM((16,), jnp.int32),
            pltpu.VMEM((WB, H), jnp.float32),
            pltpu.VMEM_SHARED((R_PAD, H), jnp.float32),
            pltpu.SemaphoreType.DMA,
            pltpu.SemaphoreType.DMA,
        ],
    )


def _sc_agg(h, srcc, dstc, kcnt):
    return _get_sc_agg()(h, srcc, dstc, kcnt)


def _embed_body(x_ref, w_ref, b_ref, o_ref):
    o_ref[...] = (
        jnp.dot(x_ref[...], w_ref[...], preferred_element_type=jnp.float32)
        + b_ref[...]
    )


def _embed(x, w, b):
    return pl.pallas_call(
        _embed_body,
        grid=(NBLK,),
        in_specs=[
            pl.BlockSpec((BLK, F), lambda i: (i, 0)),
            pl.BlockSpec((F, H), lambda i: (0, 0)),
            pl.BlockSpec((1, H), lambda i: (0, 0)),
        ],
        out_specs=pl.BlockSpec((BLK, H), lambda i: (i, 0)),
        out_shape=jax.ShapeDtypeStruct((N, H), jnp.float32),
    )(x, w, b.reshape(1, H))


def _pre_body(h_ref, a_ref, w_ref, b_ref, t_ref, s_ref):
    i = pl.program_id(0)
    u = h_ref[...] + a_ref[...]
    t = jnp.dot(u, w_ref[...], preferred_element_type=jnp.float32) + b_ref[...]
    t_ref[...] = t

    @pl.when(i == 0)
    def _():
        s_ref[...] = jnp.zeros_like(s_ref)

    s_ref[0:1, :] += jnp.sum(t, axis=0, keepdims=True)
    s_ref[1:2, :] += jnp.sum(t * t, axis=0, keepdims=True)


def _pre(h, agg, w1, b1):
    return pl.pallas_call(
        _pre_body,
        grid=(NBLK,),
        in_specs=[
            pl.BlockSpec((BLK, H), lambda i: (i, 0)),
            pl.BlockSpec((BLK, H), lambda i: (i, 0)),
            pl.BlockSpec((H, H), lambda i: (0, 0)),
            pl.BlockSpec((1, H), lambda i: (0, 0)),
        ],
        out_specs=[
            pl.BlockSpec((BLK, H), lambda i: (i, 0)),
            pl.BlockSpec((8, H), lambda i: (0, 0)),
        ],
        out_shape=[
            jax.ShapeDtypeStruct((N, H), jnp.float32),
            jax.ShapeDtypeStruct((8, H), jnp.float32),
        ],
    )(h, agg, w1, b1.reshape(1, H))


def _post_body(t_ref, s_ref, g_ref, be_ref, w_ref, b_ref, o_ref):
    m = s_ref[0:1, :] * (1.0 / N)
    v = s_ref[1:2, :] * (1.0 / N) - m * m
    scale = lax.rsqrt(v + 1e-5) * g_ref[...]
    z = jnp.maximum((t_ref[...] - m) * scale + be_ref[...], 0.0)
    o_ref[...] = jnp.maximum(
        jnp.dot(z, w_ref[...], preferred_element_type=jnp.float32) + b_ref[...],
        0.0,
    )


def _post(t, stats, g, be, w2, b2):
    return pl.pallas_call(
        _post_body,
        grid=(NBLK,),
        in_specs=[
            pl.BlockSpec((BLK, H), lambda i: (i, 0)),
            pl.BlockSpec((8, H), lambda i: (0, 0)),
            pl.BlockSpec((1, H), lambda i: (0, 0)),
            pl.BlockSpec((1, H), lambda i: (0, 0)),
            pl.BlockSpec((H, H), lambda i: (0, 0)),
            pl.BlockSpec((1, H), lambda i: (0, 0)),
        ],
        out_specs=pl.BlockSpec((BLK, H), lambda i: (i, 0)),
        out_shape=jax.ShapeDtypeStruct((N, H), jnp.float32),
    )(t, stats, g.reshape(1, H), be.reshape(1, H), w2, b2.reshape(1, H))


def _pool_body(b_ref, h1_ref, h2_ref, h3_ref, w1_ref, b1_ref, w2_ref, b2_ref,
               ge_ref, out_ref, cnt_ref):
    i = pl.program_id(0)

    @pl.when(i == 0)
    def _():
        ge_ref[...] = jnp.zeros_like(ge_ref)
        out_ref[...] = jnp.zeros_like(out_ref)
        cnt_ref[...] = jnp.zeros_like(cnt_ref)

    bt = b_ref[0, 0, :]
    onehot = (
        bt[:, None] == lax.broadcasted_iota(jnp.int32, (1, G), 1)
    ).astype(jnp.float32)
    hcat = jnp.concatenate([h1_ref[...], h2_ref[...], h3_ref[...]], axis=1)
    ge_ref[...] += lax.dot_general(
        onehot, hcat, (((0,), (0,)), ((), ())),
        preferred_element_type=jnp.float32)
    cnt_ref[...] += lax.dot_general(
        onehot, jnp.ones((BLK, 1), jnp.float32), (((0,), (0,)), ((), ())),
        preferred_element_type=jnp.float32)

    @pl.when(i == NBLK - 1)
    def _():
        ge = ge_ref[...] / jnp.maximum(cnt_ref[...], 1.0)
        ge_ref[...] = ge
        hh = jnp.maximum(
            jnp.dot(ge, w1_ref[...], preferred_element_type=jnp.float32)
            + b1_ref[...],
            0.0,
        )
        out_ref[...] = (
            jnp.dot(hh, w2_ref[...], preferred_element_type=jnp.float32)
            + b2_ref[...]
        )


def _pool_head(batch3d, h1, h2, h3, l1_W, l1_b, l2_W, l2_b):
    return pl.pallas_call(
        _pool_body,
        grid=(NBLK,),
        in_specs=[
            pl.BlockSpec((1, 1, BLK), lambda i: (i, 0, 0)),
            pl.BlockSpec((BLK, H), lambda i: (i, 0)),
            pl.BlockSpec((BLK, H), lambda i: (i, 0)),
            pl.BlockSpec((BLK, H), lambda i: (i, 0)),
            pl.BlockSpec((3 * H, 3 * H), lambda i: (0, 0)),
            pl.BlockSpec((1, 3 * H), lambda i: (0, 0)),
            pl.BlockSpec((3 * H, C), lambda i: (0, 0)),
            pl.BlockSpec((1, C), lambda i: (0, 0)),
        ],
        out_specs=[
            pl.BlockSpec((G, 3 * H), lambda i: (0, 0)),
            pl.BlockSpec((G, C), lambda i: (0, 0)),
        ],
        out_shape=[
            jax.ShapeDtypeStruct((G, 3 * H), jnp.float32),
            jax.ShapeDtypeStruct((G, C), jnp.float32),
        ],
        scratch_shapes=[pltpu.VMEM((G, 1), jnp.float32)],
    )(batch3d, h1, h2, h3, l1_W, l1_b.reshape(1, 3 * H), l2_W,
      l2_b.reshape(1, C))


def kernel(x, pos, edge_index, batch, emb_W, emb_b,
           c1_W1, c1_b1, c1_g, c1_be, c1_W2, c1_b2,
           c2_W1, c2_b1, c2_g, c2_be, c2_W2, c2_b2,
           c3_W1, c3_b1, c3_g, c3_be, c3_W2, c3_b2,
           l1_W, l1_b, l2_W, l2_b):
    src = edge_index[0]
    dst = edge_index[1]
    batch3d = batch.reshape(NBLK, 1, BLK)

    srcc, dstc, kcnt = _get_sc_partition()(src, dst)
    h = _embed(x, emb_W, emb_b)

    hs = []
    for (W1, b1, g, be, W2, b2) in (
        (c1_W1, c1_b1, c1_g, c1_be, c1_W2, c1_b2),
        (c2_W1, c2_b1, c2_g, c2_be, c2_W2, c2_b2),
        (c3_W1, c3_b1, c3_g, c3_be, c3_W2, c3_b2),
    ):
        agg = _sc_agg(h, srcc, dstc, kcnt)
        t, stats = _pre(h, agg, W1, b1)
        h = _post(t, stats, g, be, W2, b2)
        hs.append(h)

    graph_emb, out = _pool_head(batch3d, hs[0], hs[1], hs[2],
                                l1_W, l1_b, l2_W, l2_b)
    return (graph_emb, out)
